# baseline (speedup 1.0000x reference)
import jax
import jax.numpy as jnp
from jax.experimental import pallas as pl
from jax.experimental.pallas import tpu as pltpu

_BB = 1


def _lrelu(x, slope):
    return jnp.where(x >= 0, x, slope * x)


def _layer(x, xT, mj, mk, tmT, Wf3e, Wal, WarT, wae, Wec, b2):
    H, _, O = Wf3e.shape
    N = x.shape[0]
    xj = x[:mj, :]
    elT = jnp.dot(Wal, xT[:, :mj], preferred_element_type=jnp.float32)
    er = jnp.dot(x, WarT, preferred_element_type=jnp.float32)
    acc = jnp.zeros((N, O), jnp.float32)
    for h in range(H):
        ftj = jnp.dot(xj, Wf3e[h], preferred_element_type=jnp.float32)
        e = elT[h : h + 1, :] + er[:, h : h + 1] + tmT * wae[h, 0]
        e = _lrelu(e, 0.2)
        em = jnp.where(mk, e, -1e30)
        emax = jnp.max(em, axis=1, keepdims=True)
        ex = jnp.where(mk, jnp.exp(e - emax), 0.0)
        out1u = jnp.dot(ex, ftj, preferred_element_type=jnp.float32)
        esum = jnp.sum(ex, axis=1, keepdims=True)
        sx = jnp.sum(ex * tmT, axis=1, keepdims=True)
        inv = jnp.where(esum > 0, 1.0 / esum, 0.0)
        y = out1u * inv + (sx * inv) * Wec[h : h + 1, :] + b2[h : h + 1, :]
        acc = acc + _lrelu(y, 0.01)
    return acc


def _gnn_body(nf_ref, nfT_ref, mkT_ref, tmT_ref,
              Wf3_0, Wal_0, WarT_0, wae_0, We2_0, b2_0,
              Wf3_1, Wal_1, WarT_1, wae_1, We2_1, b2_1,
              Wf3_2, Wal_2, WarT_2, wae_2, We2_2, b2_2,
              out_ref):
    mj = tmT_ref.shape[2]
    N = nf_ref.shape[1]
    w0 = (Wf3_0[:], Wal_0[:], WarT_0[:], wae_0[:], We2_0[:], b2_0[:])
    w1 = (Wf3_1[:], Wal_1[:], WarT_1[:], wae_1[:], We2_1[:], b2_1[:])
    w2 = (Wf3_2[:], Wal_2[:], WarT_2[:], wae_2[:], We2_2[:], b2_2[:])
    for i in range(_BB):
        x0 = nf_ref[i]
        xT0 = nfT_ref[i]
        mk = mkT_ref[i].T > 0
        tmT = jnp.concatenate(
            [tmT_ref[i].T,
             jnp.zeros((N - mj, mj), jnp.float32)], axis=0)
        x1 = _layer(x0, xT0, mj, mk, tmT, *w0)
        x2 = _layer(x1, x1.T, mj, mk, tmT, *w1)
        x3 = _layer(x2, x2.T, mj, mk, tmT, *w2)
        out_ref[i] = x3


def _prep_weights(Wf, We, al, ar, ae, b, mj):
    H, O = al.shape
    fi = Wf.shape[0]
    Wf3 = Wf.reshape(fi, H, O).transpose(1, 0, 2)
    Wal = jnp.einsum("hio,ho->hi", Wf3, al)
    WarT = jnp.einsum("hio,ho->hi", Wf3, ar).T
    We2 = We.reshape(H, O)
    wae = jnp.sum(We2 * ae, axis=1, keepdims=True)
    return Wf3 / H, Wal, WarT, wae, We2 / H, b.reshape(H, O) / H


def kernel(Graph, norm_h, norm_L, norm_W, norm_P, norm_N, numberOfJobs,
           numberOfMachines, T, Wf0, We0, al0, ar0, ae0, b0,
           Wf1, We1, al1, ar1, ae1, b1, Wf2, We2, al2, ar2, ae2, b2):
    bs, mj, N = Graph.shape
    mm = N - mj
    H, Ofin = al2.shape

    f32 = jnp.float32
    jmask = (jnp.arange(mj)[None, :] < numberOfJobs).astype(f32)
    mmask = (jnp.arange(mm)[None, :] < numberOfMachines).astype(f32)
    jobID = jnp.arange(1, mj + 1, dtype=f32)[None, :] * jmask
    machID = jnp.arange(1, mm + 1, dtype=f32)[None, :] * mmask
    jz = jnp.zeros((bs, mj), f32)
    jobF = jnp.stack([norm_h, norm_L, jz, jz, jz, jobID, jz], axis=-1)
    mzc = jnp.zeros((bs, mm), f32)
    Wb = jnp.broadcast_to(norm_W, (bs, mm))
    Pb = jnp.broadcast_to(norm_P, (bs, mm))
    Nb = jnp.broadcast_to(norm_N, (bs, mm))
    machF = jnp.stack([mzc, mzc, Wb, Pb, Nb, mzc, machID], axis=-1)
    nf = jnp.concatenate([jobF, machF], axis=1)
    nfT = nf.transpose(0, 2, 1)

    w0 = _prep_weights(Wf0, We0, al0, ar0, ae0, b0, mj)
    w1 = _prep_weights(Wf1, We1, al1, ar1, ae1, b1, mj)
    w2 = _prep_weights(Wf2, We2, al2, ar2, ae2, b2, mj)

    def batch_spec(*dims):
        return pl.BlockSpec((_BB,) + dims, lambda bb: (bb, 0, 0))

    def full_spec(arr):
        nd = arr.ndim
        return pl.BlockSpec(arr.shape, lambda bb: (0,) * nd)

    weight_ops = list(w0) + list(w1) + list(w2)
    in_specs = ([batch_spec(N, 7), batch_spec(7, N), batch_spec(mj, N),
                 batch_spec(mj, mj)] + [full_spec(a) for a in weight_ops])

    out = pl.pallas_call(
        _gnn_body,
        grid=(bs // _BB,),
        in_specs=in_specs,
        out_specs=pl.BlockSpec((_BB, N, Ofin), lambda bb: (bb, 0, 0)),
        out_shape=jax.ShapeDtypeStruct((bs, N, Ofin), f32),
        compiler_params=pltpu.CompilerParams(
            dimension_semantics=("parallel",)),
    )(nf, nfT, Graph, T, *weight_ops)
    return out


# speedup vs baseline: 1.0565x; 1.0565x over previous
import jax
import jax.numpy as jnp
from jax.experimental import pallas as pl
from jax.experimental.pallas import tpu as pltpu

_BB = 1


def _lrelu(x, slope):
    return jnp.where(x >= 0, x, slope * x)


def _layer(x, xT, mj, mk, tmT, Wf3e, Wal, WarT, wae, Wec, b2):
    H, _, O = Wf3e.shape
    N = x.shape[0]
    xj = x[:mj, :]
    elT = jnp.dot(Wal, xT[:, :mj], preferred_element_type=jnp.float32)
    er = jnp.dot(x, WarT, preferred_element_type=jnp.float32)
    acc = jnp.zeros((N, O), jnp.float32)
    for h in range(H):
        ftj = jnp.dot(xj, Wf3e[h], preferred_element_type=jnp.float32)
        e = elT[h : h + 1, :] + er[:, h : h + 1] + tmT * wae[h, 0]
        e = _lrelu(e, 0.2)
        em = jnp.where(mk, e, -1e30)
        emax = jnp.max(em, axis=1, keepdims=True)
        ex = jnp.where(mk, jnp.exp(e - emax), 0.0)
        out1u = jnp.dot(ex, ftj, preferred_element_type=jnp.float32)
        esum = jnp.sum(ex, axis=1, keepdims=True)
        sx = jnp.sum(ex * tmT, axis=1, keepdims=True)
        inv = jnp.where(esum > 0, 1.0 / esum, 0.0)
        y = out1u * inv + (sx * inv) * Wec[h : h + 1, :] + b2[h : h + 1, :]
        acc = acc + _lrelu(y, 0.01)
    return acc


def _gnn_body(nf_ref, nfT_ref, mkT_ref, tmT_ref,
              Wf3_0, Wal_0, WarT_0, wae_0, We2_0, b2_0,
              Wf3_1, Wal_1, WarT_1, wae_1, We2_1, b2_1,
              Wf3_2, Wal_2, WarT_2, wae_2, We2_2, b2_2,
              out_ref):
    mj = tmT_ref.shape[2]
    N = nf_ref.shape[1]
    w0 = (Wf3_0[:], Wal_0[:], WarT_0[:], wae_0[:], We2_0[:], b2_0[:])
    w1 = (Wf3_1[:], Wal_1[:], WarT_1[:], wae_1[:], We2_1[:], b2_1[:])
    w2 = (Wf3_2[:], Wal_2[:], WarT_2[:], wae_2[:], We2_2[:], b2_2[:])
    for i in range(_BB):
        x0 = nf_ref[i]
        xT0 = nfT_ref[i]
        mk = mkT_ref[i] > 0.0
        tmT = tmT_ref[i]
        x1 = _layer(x0, xT0, mj, mk, tmT, *w0)
        x2 = _layer(x1, x1.T, mj, mk, tmT, *w1)
        x3 = _layer(x2, x2.T, mj, mk, tmT, *w2)
        out_ref[i] = x3


def _prep_weights(Wf, We, al, ar, ae, b, mj):
    H, O = al.shape
    fi = Wf.shape[0]
    Wf3 = Wf.reshape(fi, H, O).transpose(1, 0, 2)
    Wal = jnp.einsum("hio,ho->hi", Wf3, al)
    WarT = jnp.einsum("hio,ho->hi", Wf3, ar).T
    We2 = We.reshape(H, O)
    wae = jnp.sum(We2 * ae, axis=1, keepdims=True)
    return Wf3 / H, Wal, WarT, wae, We2 / H, b.reshape(H, O) / H


def kernel(Graph, norm_h, norm_L, norm_W, norm_P, norm_N, numberOfJobs,
           numberOfMachines, T, Wf0, We0, al0, ar0, ae0, b0,
           Wf1, We1, al1, ar1, ae1, b1, Wf2, We2, al2, ar2, ae2, b2):
    bs, mj, N = Graph.shape
    mm = N - mj
    H, Ofin = al2.shape

    f32 = jnp.float32
    jmask = (jnp.arange(mj)[None, :] < numberOfJobs).astype(f32)
    mmask = (jnp.arange(mm)[None, :] < numberOfMachines).astype(f32)
    jobID = jnp.arange(1, mj + 1, dtype=f32)[None, :] * jmask
    machID = jnp.arange(1, mm + 1, dtype=f32)[None, :] * mmask
    jz = jnp.zeros((bs, mj), f32)
    jobF = jnp.stack([norm_h, norm_L, jz, jz, jz, jobID, jz], axis=-1)
    mzc = jnp.zeros((bs, mm), f32)
    Wb = jnp.broadcast_to(norm_W, (bs, mm))
    Pb = jnp.broadcast_to(norm_P, (bs, mm))
    Nb = jnp.broadcast_to(norm_N, (bs, mm))
    machF = jnp.stack([mzc, mzc, Wb, Pb, Nb, mzc, machID], axis=-1)
    nf = jnp.concatenate([jobF, machF], axis=1)
    nfT = nf.transpose(0, 2, 1)

    maskT = (Graph != 0).astype(f32).transpose(0, 2, 1)
    tmT = jnp.pad(T, ((0, 0), (0, 0), (0, mm))).transpose(0, 2, 1)

    w0 = _prep_weights(Wf0, We0, al0, ar0, ae0, b0, mj)
    w1 = _prep_weights(Wf1, We1, al1, ar1, ae1, b1, mj)
    w2 = _prep_weights(Wf2, We2, al2, ar2, ae2, b2, mj)

    def batch_spec(*dims):
        return pl.BlockSpec((_BB,) + dims, lambda bb: (bb, 0, 0))

    def full_spec(arr):
        nd = arr.ndim
        return pl.BlockSpec(arr.shape, lambda bb: (0,) * nd)

    weight_ops = list(w0) + list(w1) + list(w2)
    in_specs = ([batch_spec(N, 7), batch_spec(7, N), batch_spec(N, mj),
                 batch_spec(N, mj)] + [full_spec(a) for a in weight_ops])

    out = pl.pallas_call(
        _gnn_body,
        grid=(bs // _BB,),
        in_specs=in_specs,
        out_specs=pl.BlockSpec((_BB, N, Ofin), lambda bb: (bb, 0, 0)),
        out_shape=jax.ShapeDtypeStruct((bs, N, Ofin), f32),
        compiler_params=pltpu.CompilerParams(
            dimension_semantics=("parallel",)),
    )(nf, nfT, maskT, tmT, *weight_ops)
    return out


# speedup vs baseline: 1.1394x; 1.0784x over previous
import jax
import jax.numpy as jnp
from jax.experimental import pallas as pl
from jax.experimental.pallas import tpu as pltpu

_BB = 4


def _lrelu(x, slope):
    return jnp.where(x >= 0, x, slope * x)


def _layer(x, xT, mj, mk, tmT, Wf3e, Wal, WarT, wae, Wec, b2):
    H, _, O = Wf3e.shape
    N = x.shape[0]
    xj = x[:mj, :]
    elT = jnp.dot(Wal, xT[:, :mj], preferred_element_type=jnp.float32)
    er = jnp.dot(x, WarT, preferred_element_type=jnp.float32)
    acc = jnp.zeros((N, O), jnp.float32)
    for h in range(H):
        ftj = jnp.dot(xj, Wf3e[h], preferred_element_type=jnp.float32)
        e = elT[h : h + 1, :] + er[:, h : h + 1] + tmT * wae[h, 0]
        e = _lrelu(e, 0.2)
        em = jnp.where(mk, e, -1e30)
        emax = jnp.max(em, axis=1, keepdims=True)
        ex = jnp.where(mk, jnp.exp(e - emax), 0.0)
        out1u = jnp.dot(ex, ftj, preferred_element_type=jnp.float32)
        esum = jnp.sum(ex, axis=1, keepdims=True)
        sx = jnp.sum(ex * tmT, axis=1, keepdims=True)
        inv = jnp.where(esum > 0, 1.0 / esum, 0.0)
        y = out1u * inv + (sx * inv) * Wec[h : h + 1, :] + b2[h : h + 1, :]
        acc = acc + _lrelu(y, 0.01)
    return acc


def _gnn_body(nf_ref, nfT_ref, mkT_ref, tmT_ref,
              Wf3_0, Wal_0, WarT_0, wae_0, We2_0, b2_0,
              Wf3_1, Wal_1, WarT_1, wae_1, We2_1, b2_1,
              Wf3_2, Wal_2, WarT_2, wae_2, We2_2, b2_2,
              out_ref):
    mj = tmT_ref.shape[2]
    N = nf_ref.shape[1]
    w0 = (Wf3_0[:], Wal_0[:], WarT_0[:], wae_0[:], We2_0[:], b2_0[:])
    w1 = (Wf3_1[:], Wal_1[:], WarT_1[:], wae_1[:], We2_1[:], b2_1[:])
    w2 = (Wf3_2[:], Wal_2[:], WarT_2[:], wae_2[:], We2_2[:], b2_2[:])
    for i in range(_BB):
        x0 = nf_ref[i]
        xT0 = nfT_ref[i]
        mk = mkT_ref[i] > 0.0
        tmT = tmT_ref[i]
        x1 = _layer(x0, xT0, mj, mk, tmT, *w0)
        x2 = _layer(x1, x1.T, mj, mk, tmT, *w1)
        x3 = _layer(x2, x2.T, mj, mk, tmT, *w2)
        out_ref[i] = x3


def _prep_weights(Wf, We, al, ar, ae, b, mj):
    H, O = al.shape
    fi = Wf.shape[0]
    Wf3 = Wf.reshape(fi, H, O).transpose(1, 0, 2)
    Wal = jnp.einsum("hio,ho->hi", Wf3, al)
    WarT = jnp.einsum("hio,ho->hi", Wf3, ar).T
    We2 = We.reshape(H, O)
    wae = jnp.sum(We2 * ae, axis=1, keepdims=True)
    return Wf3 / H, Wal, WarT, wae, We2 / H, b.reshape(H, O) / H


def kernel(Graph, norm_h, norm_L, norm_W, norm_P, norm_N, numberOfJobs,
           numberOfMachines, T, Wf0, We0, al0, ar0, ae0, b0,
           Wf1, We1, al1, ar1, ae1, b1, Wf2, We2, al2, ar2, ae2, b2):
    bs, mj, N = Graph.shape
    mm = N - mj
    H, Ofin = al2.shape

    f32 = jnp.float32
    jmask = (jnp.arange(mj)[None, :] < numberOfJobs).astype(f32)
    mmask = (jnp.arange(mm)[None, :] < numberOfMachines).astype(f32)
    jobID = jnp.arange(1, mj + 1, dtype=f32)[None, :] * jmask
    machID = jnp.arange(1, mm + 1, dtype=f32)[None, :] * mmask
    jz = jnp.zeros((bs, mj), f32)
    jobF = jnp.stack([norm_h, norm_L, jz, jz, jz, jobID, jz], axis=-1)
    mzc = jnp.zeros((bs, mm), f32)
    Wb = jnp.broadcast_to(norm_W, (bs, mm))
    Pb = jnp.broadcast_to(norm_P, (bs, mm))
    Nb = jnp.broadcast_to(norm_N, (bs, mm))
    machF = jnp.stack([mzc, mzc, Wb, Pb, Nb, mzc, machID], axis=-1)
    nf = jnp.concatenate([jobF, machF], axis=1)
    nfT = nf.transpose(0, 2, 1)

    maskT = (Graph != 0).astype(f32).transpose(0, 2, 1)
    tmT = jnp.pad(T, ((0, 0), (0, 0), (0, mm))).transpose(0, 2, 1)

    w0 = _prep_weights(Wf0, We0, al0, ar0, ae0, b0, mj)
    w1 = _prep_weights(Wf1, We1, al1, ar1, ae1, b1, mj)
    w2 = _prep_weights(Wf2, We2, al2, ar2, ae2, b2, mj)

    def batch_spec(*dims):
        return pl.BlockSpec((_BB,) + dims, lambda bb: (bb, 0, 0))

    def full_spec(arr):
        nd = arr.ndim
        return pl.BlockSpec(arr.shape, lambda bb: (0,) * nd)

    weight_ops = list(w0) + list(w1) + list(w2)
    in_specs = ([batch_spec(N, 7), batch_spec(7, N), batch_spec(N, mj),
                 batch_spec(N, mj)] + [full_spec(a) for a in weight_ops])

    out = pl.pallas_call(
        _gnn_body,
        grid=(bs // _BB,),
        in_specs=in_specs,
        out_specs=pl.BlockSpec((_BB, N, Ofin), lambda bb: (bb, 0, 0)),
        out_shape=jax.ShapeDtypeStruct((bs, N, Ofin), f32),
        compiler_params=pltpu.CompilerParams(
            dimension_semantics=("parallel",)),
    )(nf, nfT, maskT, tmT, *weight_ops)
    return out


# speedup vs baseline: 1.1737x; 1.0302x over previous
import jax
import jax.numpy as jnp
from jax.experimental import pallas as pl
from jax.experimental.pallas import tpu as pltpu

_BB = 8


def _lrelu(x, slope):
    return jnp.where(x >= 0, x, slope * x)


def _layer(x, xT, mj, mk, tmT, Wf3e, Wal, WarT, wae, Wec, b2):
    H, _, O = Wf3e.shape
    N = x.shape[0]
    xj = x[:mj, :]
    elT = jnp.dot(Wal, xT[:, :mj], preferred_element_type=jnp.float32)
    er = jnp.dot(x, WarT, preferred_element_type=jnp.float32)
    acc = jnp.zeros((N, O), jnp.float32)
    for h in range(H):
        ftj = jnp.dot(xj, Wf3e[h], preferred_element_type=jnp.float32)
        e = elT[h : h + 1, :] + er[:, h : h + 1] + tmT * wae[h, 0]
        e = _lrelu(e, 0.2)
        em = jnp.where(mk, e, -1e30)
        emax = jnp.max(em, axis=1, keepdims=True)
        ex = jnp.where(mk, jnp.exp(e - emax), 0.0)
        out1u = jnp.dot(ex, ftj, preferred_element_type=jnp.float32)
        esum = jnp.sum(ex, axis=1, keepdims=True)
        sx = jnp.sum(ex * tmT, axis=1, keepdims=True)
        inv = jnp.where(esum > 0, 1.0 / esum, 0.0)
        y = out1u * inv + (sx * inv) * Wec[h : h + 1, :] + b2[h : h + 1, :]
        acc = acc + _lrelu(y, 0.01)
    return acc


def _gnn_body(nf_ref, nfT_ref, mkT_ref, tmT_ref,
              Wf3_0, Wal_0, WarT_0, wae_0, We2_0, b2_0,
              Wf3_1, Wal_1, WarT_1, wae_1, We2_1, b2_1,
              Wf3_2, Wal_2, WarT_2, wae_2, We2_2, b2_2,
              out_ref):
    mj = tmT_ref.shape[2]
    N = nf_ref.shape[1]
    w0 = (Wf3_0[:], Wal_0[:], WarT_0[:], wae_0[:], We2_0[:], b2_0[:])
    w1 = (Wf3_1[:], Wal_1[:], WarT_1[:], wae_1[:], We2_1[:], b2_1[:])
    w2 = (Wf3_2[:], Wal_2[:], WarT_2[:], wae_2[:], We2_2[:], b2_2[:])
    for i in range(_BB):
        x0 = nf_ref[i]
        xT0 = nfT_ref[i]
        mk = mkT_ref[i] > 0.0
        tmT = tmT_ref[i]
        x1 = _layer(x0, xT0, mj, mk, tmT, *w0)
        x2 = _layer(x1, x1.T, mj, mk, tmT, *w1)
        x3 = _layer(x2, x2.T, mj, mk, tmT, *w2)
        out_ref[i] = x3


def _prep_weights(Wf, We, al, ar, ae, b, mj):
    H, O = al.shape
    fi = Wf.shape[0]
    Wf3 = Wf.reshape(fi, H, O).transpose(1, 0, 2)
    Wal = jnp.einsum("hio,ho->hi", Wf3, al)
    WarT = jnp.einsum("hio,ho->hi", Wf3, ar).T
    We2 = We.reshape(H, O)
    wae = jnp.sum(We2 * ae, axis=1, keepdims=True)
    return Wf3 / H, Wal, WarT, wae, We2 / H, b.reshape(H, O) / H


def kernel(Graph, norm_h, norm_L, norm_W, norm_P, norm_N, numberOfJobs,
           numberOfMachines, T, Wf0, We0, al0, ar0, ae0, b0,
           Wf1, We1, al1, ar1, ae1, b1, Wf2, We2, al2, ar2, ae2, b2):
    bs, mj, N = Graph.shape
    mm = N - mj
    H, Ofin = al2.shape

    f32 = jnp.float32
    jmask = (jnp.arange(mj)[None, :] < numberOfJobs).astype(f32)
    mmask = (jnp.arange(mm)[None, :] < numberOfMachines).astype(f32)
    jobID = jnp.arange(1, mj + 1, dtype=f32)[None, :] * jmask
    machID = jnp.arange(1, mm + 1, dtype=f32)[None, :] * mmask
    jz = jnp.zeros((bs, mj), f32)
    jobF = jnp.stack([norm_h, norm_L, jz, jz, jz, jobID, jz], axis=-1)
    mzc = jnp.zeros((bs, mm), f32)
    Wb = jnp.broadcast_to(norm_W, (bs, mm))
    Pb = jnp.broadcast_to(norm_P, (bs, mm))
    Nb = jnp.broadcast_to(norm_N, (bs, mm))
    machF = jnp.stack([mzc, mzc, Wb, Pb, Nb, mzc, machID], axis=-1)
    nf = jnp.concatenate([jobF, machF], axis=1)
    nfT = nf.transpose(0, 2, 1)

    maskT = (Graph != 0).astype(f32).transpose(0, 2, 1)
    tmT = jnp.pad(T, ((0, 0), (0, 0), (0, mm))).transpose(0, 2, 1)

    w0 = _prep_weights(Wf0, We0, al0, ar0, ae0, b0, mj)
    w1 = _prep_weights(Wf1, We1, al1, ar1, ae1, b1, mj)
    w2 = _prep_weights(Wf2, We2, al2, ar2, ae2, b2, mj)

    def batch_spec(*dims):
        return pl.BlockSpec((_BB,) + dims, lambda bb: (bb, 0, 0))

    def full_spec(arr):
        nd = arr.ndim
        return pl.BlockSpec(arr.shape, lambda bb: (0,) * nd)

    weight_ops = list(w0) + list(w1) + list(w2)
    in_specs = ([batch_spec(N, 7), batch_spec(7, N), batch_spec(N, mj),
                 batch_spec(N, mj)] + [full_spec(a) for a in weight_ops])

    out = pl.pallas_call(
        _gnn_body,
        grid=(bs // _BB,),
        in_specs=in_specs,
        out_specs=pl.BlockSpec((_BB, N, Ofin), lambda bb: (bb, 0, 0)),
        out_shape=jax.ShapeDtypeStruct((bs, N, Ofin), f32),
        compiler_params=pltpu.CompilerParams(
            dimension_semantics=("parallel",)),
    )(nf, nfT, maskT, tmT, *weight_ops)
    return out
